# baseline (speedup 1.0000x reference)
import functools

import jax
import jax.numpy as jnp
from jax import lax
from jax.experimental import pallas as pl
from jax.experimental.pallas import tpu as pltpu
from jax.experimental.pallas import tpu_sc as plsc

_B = 16384
_K = 64
_N = 1000000
_HALF = 500096
_SEG = _HALF // 16
_SPLIT = 499904
_BPW = _B // 16
_NBUF = 3
_PHASES = _K * 2


def _build_lists(idw_u, idw_v, cmask, loclist, blist, cnts, half):
    def step(j, cur):
        off = pl.multiple_of(j * 16, 16)
        ids = jnp.where(cmask, idw_u[pl.ds(off, 16)], idw_v[pl.ds(off, 16)])
        if half == 0:
            mask = ids < _SPLIT
            loc = ids
        else:
            mask = ids >= _SPLIT
            loc = ids - _SPLIT
        bpos = lax.iota(jnp.int32, 16) + off
        plsc.store_compressed(loclist.at[pl.ds(cur, 16)], loc, mask=mask)
        plsc.store_compressed(blist.at[pl.ds(cur, 16)], bpos, mask=mask)
        cnt = plsc.all_reduce_population_count(mask)
        return cur + cnt[0]

    n = lax.fori_loop(0, _BPW // 16, step, 0)
    cnts[pl.ds(half * 16, 16)] = jnp.full((16,), 1, jnp.int32) * n


def _sc_stage1(uid_hbm, mid_hbm, ut_hbm, vt_hbm, uval_hbm, vval_hbm,
               idw_u, idw_v, loclist0, blist0, loclist1, blist1,
               valbuf, valk, shared, cnts, sem, gsem, dsem):
    c = lax.axis_index("c")
    s = lax.axis_index("s")

    pltpu.sync_copy(uid_hbm.at[pl.ds(s * _BPW, _BPW)], idw_u)
    pltpu.sync_copy(mid_hbm.at[pl.ds(s * _BPW, _BPW)], idw_v)
    cmask = (jnp.full((16,), 1, jnp.int32) * c) == 0

    def zi(j, carry):
        off = pl.multiple_of(j * 16, 16)
        z = jnp.zeros((16,), jnp.int32)
        loclist0[pl.ds(off, 16)] = z
        loclist1[pl.ds(off, 16)] = z
        return carry

    lax.fori_loop(0, (_BPW + 16) // 16, zi, 0)

    _build_lists(idw_u, idw_v, cmask, loclist0, blist0, cnts, 0)
    _build_lists(idw_u, idw_v, cmask, loclist1, blist1, cnts, 1)

    def stream(p, buf):
        h = p & 1
        k = p >> 1
        src = pl.ds(h * _SPLIT + s * _SEG, _SEG)
        dst = pl.ds(buf * _HALF + s * _SEG, _SEG)

        @pl.when(c == 0)
        def _():
            pltpu.make_async_copy(ut_hbm.at[k, src], shared.at[dst], sem.at[buf]).start()

        @pl.when(c == 1)
        def _():
            pltpu.make_async_copy(vt_hbm.at[k, src], shared.at[dst], sem.at[buf]).start()

    def stream_wait(buf):
        dst = pl.ds(buf * _HALF + s * _SEG, _SEG)
        pltpu.make_async_copy(ut_hbm.at[0, pl.ds(0, _SEG)], shared.at[dst],
                              sem.at[buf]).wait()

    def _extract_half(loclist, blist, nvec, buf):
        base = buf * _HALF
        for cc in range(_BPW // 128):
            pltpu.make_async_copy(
                shared.at[pl.ds(base, _HALF)].at[loclist.at[pl.ds(cc * 128, 128)]],
                valbuf.at[pl.ds(cc * 128, 128)], gsem).start()
        for cc in range(_BPW // 128):
            pltpu.make_async_copy(
                shared.at[pl.ds(base, _HALF)].at[loclist.at[pl.ds(cc * 128, 128)]],
                valbuf.at[pl.ds(cc * 128, 128)], gsem).wait()

        def place(j, carry):
            off = pl.multiple_of(j * 16, 16)
            sl = pl.ds(off, 16)
            bvec = blist[sl]
            vals = valbuf[sl]
            mask = (lax.iota(jnp.int32, 16) + off) < nvec
            plsc.store_scatter(valk, [bvec], vals, mask=mask)
            return carry

        lax.fori_loop(0, _BPW // 16, place, 0)

    def dump(k):
        dst = pl.ds(pl.multiple_of(s * _BPW, 8), _BPW)

        @pl.when(c == 0)
        def _():
            pltpu.make_async_copy(valk, uval_hbm.at[k, dst], dsem).start()

        @pl.when(c == 1)
        def _():
            pltpu.make_async_copy(valk, vval_hbm.at[k, dst], dsem).start()

        pltpu.make_async_copy(valk, uval_hbm.at[k, dst], dsem).wait()

    stream(0, 0)

    def phase(p, carry):
        buf = lax.rem(p, _NBUF)

        @pl.when(p + 1 < _PHASES)
        def _():
            stream(p + 1, lax.rem(p + 1, _NBUF))

        stream_wait(buf)
        plsc.subcore_barrier()

        h = p & 1

        @pl.when(h == 0)
        def _():
            _extract_half(loclist0, blist0, cnts[pl.ds(0, 16)], buf)

        @pl.when(h == 1)
        def _():
            _extract_half(loclist1, blist1, cnts[pl.ds(16, 16)], buf)
            dump(p >> 1)

        return carry

    lax.fori_loop(0, _PHASES, phase, 0)


def _tc_dot(u_ref, v_ref, o_ref):
    o_ref[...] = jnp.sum(u_ref[...] * v_ref[...], axis=0)


@jax.jit
def kernel(uid, mid, u_emb, i_emb):
    mesh = plsc.VectorSubcoreMesh(core_axis_name="c", subcore_axis_name="s")
    stage1 = functools.partial(
        pl.kernel,
        mesh=mesh,
        out_type=(jax.ShapeDtypeStruct((_K, _B), jnp.float32),
                  jax.ShapeDtypeStruct((_K, _B), jnp.float32)),
        scratch_types=[
            pltpu.VMEM((_BPW,), jnp.int32),
            pltpu.VMEM((_BPW,), jnp.int32),
            pltpu.VMEM((_BPW + 16,), jnp.int32),
            pltpu.VMEM((_BPW + 16,), jnp.int32),
            pltpu.VMEM((_BPW + 16,), jnp.int32),
            pltpu.VMEM((_BPW + 16,), jnp.int32),
            pltpu.VMEM((_BPW,), jnp.float32),
            pltpu.VMEM((_BPW,), jnp.float32),
            pltpu.VMEM_SHARED((_NBUF * _HALF,), jnp.float32),
            pltpu.VMEM((32,), jnp.int32),
            pltpu.SemaphoreType.DMA((_NBUF,)),
            pltpu.SemaphoreType.DMA,
            pltpu.SemaphoreType.DMA,
        ],
        compiler_params=pltpu.CompilerParams(
            use_tc_tiling_on_sc=False, needs_layout_passes=False),
    )(_sc_stage1)
    uval, vval = stage1(uid, mid, u_emb.T, i_emb.T)

    out = pl.pallas_call(
        _tc_dot,
        out_shape=jax.ShapeDtypeStruct((_B,), jnp.float32),
        grid=(16,),
        in_specs=[
            pl.BlockSpec((_K, _B // 16), lambda j: (0, j)),
            pl.BlockSpec((_K, _B // 16), lambda j: (0, j)),
        ],
        out_specs=pl.BlockSpec((_B // 16,), lambda j: (j,)),
    )(uval, vval)
    return out


# speedup vs baseline: 1.0236x; 1.0236x over previous
import functools

import jax
import jax.numpy as jnp
from jax import lax
from jax.experimental import pallas as pl
from jax.experimental.pallas import tpu as pltpu
from jax.experimental.pallas import tpu_sc as plsc

_B = 16384
_K = 64
_SEG = 31256
_HALF = _SEG * 16
_SPLIT = 499904
_VCAP = _B + 128
_LCAP = _B + 16
_PHASES = _K * 2


def _sc_stage1(uid_hbm, mid_hbm, ut_hbm, vt_hbm, uval_hbm, vval_hbm,
               idbuf, lists, nlist, chunk, sval, sidx, val_sp,
               ssem, gsem, dsem, isem):
    c = lax.axis_index("c")
    s = lax.axis_index("s")
    iota16 = lax.iota(jnp.int32, 16)

    @pl.when(c == 0)
    def _():
        pltpu.make_async_copy(uid_hbm, idbuf, isem).start()

    @pl.when(c == 1)
    def _():
        pltpu.make_async_copy(mid_hbm, idbuf, isem).start()

    pltpu.make_async_copy(uid_hbm, idbuf, isem).wait()

    lo0 = s * _SEG
    hi0 = jnp.minimum(lo0 + _SEG, _SPLIT)
    lo1 = _SPLIT + s * _SEG
    hi1 = jnp.minimum(lo1 + _SEG, 1000000)

    def build(half, lo, hi):
        lov = jnp.full((16,), 1, jnp.int32) * lo
        hiv = jnp.full((16,), 1, jnp.int32) * hi

        def step(j, cur):
            off = pl.multiple_of(j * 16, 16)
            ids = idbuf[pl.ds(off, 16)]
            mask = (ids >= lov) & (ids < hiv)
            packed = (ids - lov) | ((iota16 + off) << 15)
            plsc.store_compressed(lists.at[half].at[pl.ds(cur, 16)], packed,
                                  mask=mask)
            cnt = plsc.all_reduce_population_count(mask)
            return cur + cnt[0]

        n = lax.fori_loop(0, _B // 16, step, 0)
        nlist[pl.ds(half * 16, 16)] = jnp.full((16,), 1, jnp.int32) * n

    build(0, lo0, hi0)
    build(1, lo1, hi1)

    def stream(p, slot):
        h = p & 1
        k = p >> 1
        src = pl.ds(h * _SPLIT + s * _SEG, _SEG)

        @pl.when((c == 0) & (p < _PHASES))
        def _():
            pltpu.make_async_copy(ut_hbm.at[k, src], chunk.at[slot],
                                  ssem.at[slot]).start()

        @pl.when((c == 1) & (p < _PHASES))
        def _():
            pltpu.make_async_copy(vt_hbm.at[k, src], chunk.at[slot],
                                  ssem.at[slot]).start()

    def stream_wait(slot):
        pltpu.make_async_copy(ut_hbm.at[0, pl.ds(0, _SEG)], chunk.at[slot],
                              ssem.at[slot]).wait()

    def extract(p, slot):
        h = p & 1
        k = p >> 1
        vbase = (k & 1) * _VCAP
        nvec = nlist[pl.ds(pl.multiple_of(h * 16, 16), 16)]
        ngroups = jnp.maximum((nvec[0] + 511) >> 9, 1)
        slotvec = jnp.full((16,), 1, jnp.int32) * slot
        vbasev = jnp.full((16,), 1, jnp.int32) * vbase

        def block(g, ring):
            boff = pl.multiple_of(g * 512, 128) + ring * 128
            for v in range(8):
                goff = boff + v * 16
                lm = (iota16 + goff) < nvec
                pk = lists[h, pl.ds(goff, 16)]
                loc = jnp.where(lm, pk & 32767, 0)
                bb = jnp.where(lm, pk >> 15, _B) + vbasev
                vals = plsc.load_gather(chunk, [slotvec, loc])
                sval[ring, pl.ds(v * 16, 16)] = vals
                sidx[ring, pl.ds(v * 16, 16)] = bb
            pltpu.make_async_copy(
                sval.at[ring], val_sp.at[sidx.at[ring]], gsem.at[ring]).start()

        def group0(g, carry):
            for r in range(4):
                block(g, r)
            return carry

        def group(g, carry):
            for r in range(4):
                pltpu.make_async_copy(
                    sval.at[r], val_sp.at[sidx.at[r]], gsem.at[r]).wait()
                block(g, r)
            return carry

        lax.fori_loop(0, 1, group0, 0)
        lax.fori_loop(1, ngroups, group, 0)

        for r in range(4):
            pltpu.make_async_copy(
                sval.at[r], val_sp.at[sidx.at[r]], gsem.at[r]).wait()

    def dump(k):
        vbase = (k & 1) * _VCAP
        src = pl.ds(vbase + s * 1024, 1024)
        dst = pl.ds(pl.multiple_of(s * 1024, 8), 1024)

        @pl.when(c == 0)
        def _():
            pltpu.make_async_copy(val_sp.at[src], uval_hbm.at[k, dst],
                                  dsem).start()

        @pl.when(c == 1)
        def _():
            pltpu.make_async_copy(val_sp.at[src], vval_hbm.at[k, dst],
                                  dsem).start()

        pltpu.make_async_copy(val_sp.at[src], uval_hbm.at[k, dst], dsem).wait()

    stream(0, 0)

    def one_phase(p, slot):
        stream(p + 1, 1 - slot)
        stream_wait(slot)
        extract(p, slot)
        plsc.subcore_barrier()
        dump(p >> 1)

    def phase_pair(t, carry):
        one_phase(t * 2, 0)
        one_phase(t * 2 + 1, 1)
        return carry

    lax.fori_loop(0, _PHASES // 2, phase_pair, 0)


def _tc_dot(u_ref, v_ref, o_ref):
    o_ref[...] = jnp.sum(u_ref[...] * v_ref[...], axis=0)


@jax.jit
def kernel(uid, mid, u_emb, i_emb):
    mesh = plsc.VectorSubcoreMesh(core_axis_name="c", subcore_axis_name="s")
    stage1 = functools.partial(
        pl.kernel,
        mesh=mesh,
        out_type=(jax.ShapeDtypeStruct((_K, _B), jnp.float32),
                  jax.ShapeDtypeStruct((_K, _B), jnp.float32)),
        scratch_types=[
            pltpu.VMEM((_B,), jnp.int32),
            pltpu.VMEM((2, _LCAP), jnp.int32),
            pltpu.VMEM((32,), jnp.int32),
            pltpu.VMEM((2, _SEG), jnp.float32),
            pltpu.VMEM((4, 128), jnp.float32),
            pltpu.VMEM((4, 128), jnp.int32),
            pltpu.VMEM_SHARED((2 * _VCAP,), jnp.float32),
            pltpu.SemaphoreType.DMA((2,)),
            pltpu.SemaphoreType.DMA((4,)),
            pltpu.SemaphoreType.DMA,
            pltpu.SemaphoreType.DMA,
        ],
        compiler_params=pltpu.CompilerParams(
            use_tc_tiling_on_sc=False, needs_layout_passes=False),
    )(_sc_stage1)
    uval, vval = stage1(uid, mid, u_emb.T, i_emb.T)

    out = pl.pallas_call(
        _tc_dot,
        out_shape=jax.ShapeDtypeStruct((_B,), jnp.float32),
        grid=(16,),
        in_specs=[
            pl.BlockSpec((_K, _B // 16), lambda j: (0, j)),
            pl.BlockSpec((_K, _B // 16), lambda j: (0, j)),
        ],
        out_specs=pl.BlockSpec((_B // 16,), lambda j: (j,)),
    )(uval, vval)
    return out
